# baseline (speedup 1.0000x reference)
import functools
import jax
import jax.numpy as jnp
from jax import lax
from jax.experimental import pallas as pl
from jax.experimental.pallas import tpu as pltpu
from jax.experimental.pallas import tpu_sc as plsc

NN = 10000
EE = 320000
HH = 128
CC = 10
GG = 128
EPS = 1e-5

NC, NS, LN = 2, 16, 16
NW = NC * NS
NP = 10240
NPT = NP // NS
EPT = 10240
EPAD = EPT * NW
CH = 64
NSTEP = EPT // CH
NBUF = 2
NBUFC = 4
NP2 = 10112
NPT2 = NP2 // NS
DUMMY = NN + 8

_MESH = plsc.VectorSubcoreMesh(core_axis_name="c", subcore_axis_name="s")

F32 = jnp.float32
I32 = jnp.int32


def _fill1d(ref, n, value):
    def body(j, carry):
        ref[pl.ds(j * LN, LN)] = jnp.full((LN,), value, F32)
        return carry
    lax.fori_loop(0, n // LN, body, 0)


@functools.partial(
    pl.kernel,
    out_type=jax.ShapeDtypeStruct((NC, NP), F32),
    mesh=_MESH,
    compiler_params=pltpu.CompilerParams(needs_layout_passes=False),
    scratch_types=[
        pltpu.VMEM((CH,), I32),
        pltpu.VMEM((CH,), F32),
        pltpu.VMEM((NPT,), F32),
        pltpu.VMEM_SHARED((NP,), F32),
    ],
)
def _sc_outdeg(row_hbm, out_hbm, idx_v, ones_v, zs_v, acc):
    cid = lax.axis_index("c")
    sid = lax.axis_index("s")
    wid = sid * NC + cid
    base = wid * EPT
    _fill1d(ones_v, CH, 1.0)
    _fill1d(zs_v, NPT, 0.0)
    pltpu.sync_copy(zs_v, acc.at[pl.ds(sid * NPT, NPT)])
    plsc.subcore_barrier()

    def step(s, carry):
        pltpu.sync_copy(row_hbm.at[pl.ds(base + s * CH, CH)], idx_v)
        pltpu.sync_copy(ones_v, acc.at[idx_v], add=True)
        return carry
    lax.fori_loop(0, NSTEP, step, 0)
    plsc.subcore_barrier()
    pltpu.sync_copy(acc.at[pl.ds(sid * NPT, NPT)],
                    out_hbm.at[cid, pl.ds(sid * NPT, NPT)])


def _sc_conv(ys_hbm, row_hbm, col_hbm, zeros_hbm, out_hbm,
             rowall_v, cidx_v, rows_v, sems, csems, acc):
    cid = lax.axis_index("c")
    sid = lax.axis_index("s")
    wid = sid * NC + cid
    base = wid * EPT
    pltpu.sync_copy(row_hbm.at[pl.ds(base, EPT)], rowall_v)
    pltpu.sync_copy(zeros_hbm.at[pl.ds(sid * NPT2, NPT2), :],
                    acc.at[pl.ds(sid * NPT2, NPT2), :])
    plsc.subcore_barrier()

    def step(s4, carry):
        descs = []
        cdescs = []
        for b in range(NBUFC):
            s = s4 * NBUFC + b
            cdescs.append(pltpu.async_copy(
                col_hbm.at[pl.ds(base + s * CH, CH)], cidx_v[b], csems[b]))
            descs.append(pltpu.async_copy(
                ys_hbm.at[rowall_v.at[pl.ds(s * CH, CH)]],
                rows_v[b], sems[b]))
        for b in range(NBUFC):
            cdescs[b].wait()
            descs[b].wait()
            pltpu.sync_copy(rows_v[b], acc.at[cidx_v[b]], add=True)
        return carry
    lax.fori_loop(0, NSTEP // NBUFC, step, 0)
    plsc.subcore_barrier()
    pltpu.sync_copy(acc.at[pl.ds(sid * NPT2, NPT2), :],
                    out_hbm.at[cid, pl.ds(sid * NPT2, NPT2), :])


_sc_conv = functools.partial(
    pl.kernel,
    out_type=jax.ShapeDtypeStruct((NC, NP2, HH), F32),
    mesh=_MESH,
    compiler_params=pltpu.CompilerParams(needs_layout_passes=False),
    scratch_types=[
        pltpu.VMEM((EPT,), I32),
        [pltpu.VMEM((CH,), I32) for _ in range(NBUFC)],
        [pltpu.VMEM((CH, HH), F32) for _ in range(NBUFC)],
        [pltpu.SemaphoreType.DMA for _ in range(NBUFC)],
        [pltpu.SemaphoreType.DMA for _ in range(NBUFC)],
        pltpu.VMEM_SHARED((NP2, HH), F32),
    ],
)(_sc_conv)


@functools.partial(
    pl.kernel,
    out_type=(jax.ShapeDtypeStruct((EPAD,), F32),
              jax.ShapeDtypeStruct((NC, NP), F32)),
    mesh=_MESH,
    compiler_params=pltpu.CompilerParams(needs_layout_passes=False),
    scratch_types=[
        pltpu.VMEM((NP,), F32),
        pltpu.VMEM((NP,), F32),
        pltpu.VMEM((EPT,), I32),
        pltpu.VMEM((EPT,), I32),
        pltpu.VMEM((EPT,), F32),
        pltpu.VMEM((CH,), I32),
        pltpu.VMEM((NPT,), F32),
        pltpu.VMEM_SHARED((NP,), F32),
    ],
)
def _sc_edge_att(da_hbm, db_hbm, row_hbm, col_hbm, ew_hbm, deg_hbm,
                 da_v, db_v, row_v, col_v, ew_v, idx_v, zs_v, acc):
    cid = lax.axis_index("c")
    sid = lax.axis_index("s")
    wid = sid * NC + cid
    base = wid * EPT
    _fill1d(zs_v, NPT, 0.0)
    pltpu.sync_copy(zs_v, acc.at[pl.ds(sid * NPT, NPT)])
    pltpu.sync_copy(da_hbm, da_v.at[pl.ds(0, NN)])
    pltpu.sync_copy(db_hbm, db_v.at[pl.ds(0, NN)])
    pltpu.sync_copy(row_hbm.at[pl.ds(base, EPT)], row_v)
    pltpu.sync_copy(col_hbm.at[pl.ds(base, EPT)], col_v)
    plsc.subcore_barrier()

    def step(j, carry):
        r16 = row_v[pl.ds(j * LN, LN)]
        c16 = col_v[pl.ds(j * LN, LN)]
        va = plsc.load_gather(da_v, [r16])
        vb = plsc.load_gather(db_v, [c16])
        ew = 1.0 / (1.0 + jnp.exp(-(va + vb)))
        ew_v[pl.ds(j * LN, LN)] = ew
        return carry
    lax.fori_loop(0, EPT // LN, step, 0)
    pltpu.sync_copy(ew_v, ew_hbm.at[pl.ds(base, EPT)])

    def step2(s, carry):
        pltpu.sync_copy(row_hbm.at[pl.ds(base + s * CH, CH)], idx_v)
        pltpu.sync_copy(ew_v.at[pl.ds(s * CH, CH)], acc.at[idx_v], add=True)
        return carry
    lax.fori_loop(0, NSTEP, step2, 0)
    plsc.subcore_barrier()
    pltpu.sync_copy(acc.at[pl.ds(sid * NPT, NPT)],
                    deg_hbm.at[cid, pl.ds(sid * NPT, NPT)])


def _make_sc_wconv(is_ctx):
    def body(z_hbm, ew_hbm, row_hbm, col_hbm, zeros_hbm, q_hbm,
             rowall_v, cidx_v, ew_v, rows_v, sems, csems, acc):
        cid = lax.axis_index("c")
        sid = lax.axis_index("s")
        wid = sid * NC + cid
        base = wid * EPT
        pltpu.sync_copy(row_hbm.at[pl.ds(base, EPT)], rowall_v)
        pltpu.sync_copy(ew_hbm.at[pl.ds(base, EPT)], ew_v)
        pltpu.sync_copy(zeros_hbm.at[pl.ds(sid * NPT2, NPT2), :],
                        acc.at[pl.ds(sid * NPT2, NPT2), :])
        plsc.subcore_barrier()

        def step(s4, carry):
            descs = []
            cdescs = []
            for b in range(NBUF):
                s = s4 * NBUF + b
                cdescs.append(pltpu.async_copy(
                    col_hbm.at[pl.ds(base + s * CH, CH)], cidx_v[b],
                    csems[b]))
                descs.append(pltpu.async_copy(
                    z_hbm.at[rowall_v.at[pl.ds(s * CH, CH)]],
                    rows_v[b], sems[b]))
            for b in range(NBUF):
                s = s4 * NBUF + b
                cdescs[b].wait()
                descs[b].wait()

                def scale(i16, c2):
                    w16 = ew_v[pl.ds(s * CH + i16 * LN, LN)]
                    if not is_ctx:
                        w16 = 1.0 - w16
                    for l in range(LN):
                        w = w16[l]
                        r = i16 * LN + l
                        for j in range(HH // LN):
                            rows_v[b][r, pl.ds(j * LN, LN)] = (
                                w * rows_v[b][r, pl.ds(j * LN, LN)])
                    return c2
                lax.fori_loop(0, CH // LN, scale, 0)
                pltpu.sync_copy(rows_v[b], acc.at[cidx_v[b]], add=True)
            return carry
        lax.fori_loop(0, NSTEP // NBUF, step, 0)
        plsc.subcore_barrier()
        pltpu.sync_copy(acc.at[pl.ds(sid * NPT2, NPT2), :],
                        q_hbm.at[cid, pl.ds(sid * NPT2, NPT2), :])

    return pl.kernel(
        body,
        out_type=jax.ShapeDtypeStruct((NC, NP2, HH), F32),
        mesh=_MESH,
        compiler_params=pltpu.CompilerParams(needs_layout_passes=False),
        scratch_types=[
            pltpu.VMEM((EPT,), I32),
            [pltpu.VMEM((CH,), I32) for _ in range(NBUF)],
            pltpu.VMEM((EPT,), F32),
            [pltpu.VMEM((CH, HH), F32) for _ in range(NBUF)],
            [pltpu.SemaphoreType.DMA for _ in range(NBUF)],
            [pltpu.SemaphoreType.DMA for _ in range(NBUF)],
            pltpu.VMEM_SHARED((NP2, HH), F32),
        ],
    )


_sc_wconv_c = _make_sc_wconv(True)
_sc_wconv_o = _make_sc_wconv(False)


def _bn(x):
    mean = jnp.mean(x, axis=0, keepdims=True)
    var = jnp.mean((x - mean) ** 2, axis=0, keepdims=True)
    return (x - mean) / jnp.sqrt(var + EPS) * 1.0 + 0.0001


def _mm(a, b):
    return lax.dot_general(a, b, (((1,), (0,)), ((), ())),
                           precision=lax.Precision.HIGHEST,
                           preferred_element_type=F32)


def _tc_call(body, out_shapes):
    return pl.pallas_call(
        body,
        out_shape=out_shapes,
    )


def _tc_feat_body(x_ref, w_ref, b_ref, h_ref):
    x = _bn(x_ref[...])
    h_ref[...] = jax.nn.relu(_mm(x, w_ref[...]) + b_ref[...][None, :])


def _tc_prep0_body(h_ref, w_ref, degp_ref, ys_ref, dinv_ref, outdeg_ref):
    outdeg = degp_ref[0, :NN] + degp_ref[1, :NN]
    dinv = lax.rsqrt(outdeg + 1.0)
    dinv_ref[...] = dinv[:, None]
    outdeg_ref[...] = outdeg[:, None]
    xb = _bn(h_ref[...])
    ys_ref[...] = dinv[:, None] * _mm(xb, w_ref[...])


def _tc_prep_body(p_ref, ysp_ref, dinv_ref, bprev_ref, w_ref, ys_ref):
    dinv = dinv_ref[...]
    out = dinv * (p_ref[0, :NN, :] + p_ref[1, :NN, :] + ysp_ref[...]) \
        + bprev_ref[...][None, :]
    h = jax.nn.relu(out)
    xb = _bn(h)
    ys_ref[...] = dinv * _mm(xb, w_ref[...])


def _tc_finish_body(p_ref, ysp_ref, dinv_ref, bprev_ref, x_ref):
    out = dinv_ref[...] * (p_ref[0, :NN, :] + p_ref[1, :NN, :]
                           + ysp_ref[...]) + bprev_ref[...][None, :]
    x_ref[...] = jax.nn.relu(out)


def _tc_edge_proj_body(x_ref, wea_ref, bea_ref, da_ref, db_ref):
    x = x_ref[...]
    wea = wea_ref[...]
    wd_a = wea[:HH, 0:1] - wea[:HH, 1:2]
    wd_b = wea[HH:, 0:1] - wea[HH:, 1:2]
    bea = bea_ref[...]
    da_ref[...] = _mm(x, wd_a) + (bea[0] - bea[1])
    db_ref[...] = _mm(x, wd_b)


def _tc_node_att_body(x_ref, wna_ref, bna_ref, xc_ref, xo_ref):
    x = x_ref[...]
    nl = _mm(x, wna_ref[...]) + bna_ref[...][None, :]
    na0 = jax.nn.sigmoid(nl[:, 0:1] - nl[:, 1:2])
    xc_ref[...] = na0 * x
    xo_ref[...] = (1.0 - na0) * x


def _make_wprep_body(is_ctx):
    def body(x_ref, w_ref, degcp_ref, outdeg_ref, z_ref, dinv_ref):
        degc = degcp_ref[0, :NN] + degcp_ref[1, :NN] + 1.0
        if is_ctx:
            deg = degc
        else:
            deg = outdeg_ref[...][:, 0] + 2.0 - degc
        dinv = lax.rsqrt(deg)[:, None]
        dinv_ref[...] = dinv
        z_ref[...] = dinv * _mm(_bn(x_ref[...]), w_ref[...])
    return body


def _readout(h, w1, b1, w2, b2):
    h = _bn(h)
    h = jax.nn.relu(_mm(h, w1) + b1[None, :])
    h = _bn(h)
    h = _mm(h, w2) + b2[None, :]
    m = jnp.max(h, axis=-1, keepdims=True)
    lse = jnp.log(jnp.sum(jnp.exp(h - m), axis=-1, keepdims=True)) + m
    return h - lse


def _tc_pool_body(q_ref, z_ref, dinv_ref, b_ref, batch_ref, p_ref):
    xr = jax.nn.relu(
        dinv_ref[...] * (q_ref[0, :NN, :] + q_ref[1, :NN, :] + z_ref[...])
        + b_ref[...][None, :])
    gi = lax.broadcasted_iota(I32, (1, GG), 1)
    oh = (batch_ref[...] == gi).astype(F32)
    p_ref[...] = lax.dot_general(oh, xr, (((0,), (0,)), ((), ())),
                                 precision=lax.Precision.HIGHEST,
                                 preferred_element_type=F32)


def _tc_readout_body(pc_ref, po_ref,
                     w1c_ref, b1c_ref, w2c_ref, b2c_ref,
                     w1o_ref, b1o_ref, w2o_ref, b2o_ref,
                     w1co_ref, b1co_ref, w2co_ref, b2co_ref,
                     lc_ref, lo_ref, lco_ref):
    pc = pc_ref[...]
    po = po_ref[...]
    lc_ref[...] = _readout(pc, w1c_ref[...], b1c_ref[...],
                           w2c_ref[...], b2c_ref[...])
    lo_ref[...] = _readout(po, w1o_ref[...], b1o_ref[...],
                           w2o_ref[...], b2o_ref[...])
    lco_ref[...] = _readout(pc + po, w1co_ref[...], b1co_ref[...],
                            w2co_ref[...], b2co_ref[...])


def kernel(x, params, edge_index, batch):
    row = edge_index[0]
    col = edge_index[1]
    pad = EPAD - EE
    row_p = jnp.concatenate([row, jnp.zeros((pad,), I32)])
    rowd_p = jnp.concatenate([row, jnp.full((pad,), DUMMY, I32)])
    col_p = jnp.concatenate([col, jnp.full((pad,), DUMMY, I32)])
    zeros2 = jnp.zeros((NP2, HH), F32)
    sds = jax.ShapeDtypeStruct

    degp = _sc_outdeg(rowd_p)

    h = _tc_call(_tc_feat_body, sds((NN, HH), F32))(
        x, params["W_feat"], params["b_feat"])

    ys, dinv, outdeg = _tc_call(
        _tc_prep0_body,
        (sds((NN, HH), F32), sds((NN, 1), F32), sds((NN, 1), F32)))(
        h, params["W_convs"][0], degp)

    for i in range(1, 4):
        part = _sc_conv(ys, row_p, col_p, zeros2)
        if i < 3:
            ys = _tc_call(_tc_prep_body, sds((NN, HH), F32))(
                part, ys, dinv, params["b_convs"][i - 1],
                params["W_convs"][i])

    xatt = _tc_call(_tc_finish_body, sds((NN, HH), F32))(
        part, ys, dinv, params["b_convs"][2])

    da, db = _tc_call(
        _tc_edge_proj_body, (sds((NN, 1), F32), sds((NN, 1), F32)))(
        xatt, params["W_edge_att"], params["b_edge_att"])
    xc, xo = _tc_call(
        _tc_node_att_body, (sds((NN, HH), F32), sds((NN, HH), F32)))(
        xatt, params["W_node_att"], params["b_node_att"])

    ewc, degcp = _sc_edge_att(da[:, 0], db[:, 0], rowd_p, col_p)

    zc, dinvc = _tc_call(
        _make_wprep_body(True), (sds((NN, HH), F32), sds((NN, 1), F32)))(
        xc, params["W_ctx"], degcp, outdeg)
    zo, dinvo = _tc_call(
        _make_wprep_body(False), (sds((NN, HH), F32), sds((NN, 1), F32)))(
        xo, params["W_obj"], degcp, outdeg)

    qc = _sc_wconv_c(zc, ewc, row_p, col_p, zeros2)
    qo = _sc_wconv_o(zo, ewc, row_p, col_p, zeros2)

    batch2 = batch[:, None]
    pool = _tc_call(_tc_pool_body, sds((GG, HH), F32))
    pc = pool(qc, zc, dinvc, params["b_ctx"], batch2)
    po = pool(qo, zo, dinvo, params["b_obj"], batch2)

    lc, lo, lco = _tc_call(
        _tc_readout_body,
        (sds((GG, CC), F32), sds((GG, CC), F32), sds((GG, CC), F32)))(
        pc, po,
        params["W_fc1_c"], params["b_fc1_c"],
        params["W_fc2_c"], params["b_fc2_c"],
        params["W_fc1_o"], params["b_fc1_o"],
        params["W_fc2_o"], params["b_fc2_o"],
        params["W_fc1_co"], params["b_fc1_co"],
        params["W_fc2_co"], params["b_fc2_co"])
    return lc, lo, lco


# speedup vs baseline: 1.0201x; 1.0201x over previous
import functools
import jax
import jax.numpy as jnp
from jax import lax
from jax.experimental import pallas as pl
from jax.experimental.pallas import tpu as pltpu
from jax.experimental.pallas import tpu_sc as plsc

NN = 10000
EE = 320000
HH = 128
CC = 10
GG = 128
EPS = 1e-5

NC, NS, LN = 2, 16, 16
NW = NC * NS
NP = 10240
NPT = NP // NS
EPT = 10240
EPAD = EPT * NW
CH = 128
NSTEP = EPT // CH
NBUF = 2
NBUFC = 2
NP2 = 10112
NPT2 = NP2 // NS
DUMMY = NN + 8

_MESH = plsc.VectorSubcoreMesh(core_axis_name="c", subcore_axis_name="s")

F32 = jnp.float32
I32 = jnp.int32


def _fill1d(ref, n, value):
    def body(j, carry):
        ref[pl.ds(j * LN, LN)] = jnp.full((LN,), value, F32)
        return carry
    lax.fori_loop(0, n // LN, body, 0)


@functools.partial(
    pl.kernel,
    out_type=jax.ShapeDtypeStruct((NC, NP), F32),
    mesh=_MESH,
    compiler_params=pltpu.CompilerParams(needs_layout_passes=False),
    scratch_types=[
        pltpu.VMEM((CH,), I32),
        pltpu.VMEM((CH,), F32),
        pltpu.VMEM((NPT,), F32),
        pltpu.VMEM_SHARED((NP,), F32),
    ],
)
def _sc_outdeg(row_hbm, out_hbm, idx_v, ones_v, zs_v, acc):
    cid = lax.axis_index("c")
    sid = lax.axis_index("s")
    wid = sid * NC + cid
    base = wid * EPT
    _fill1d(ones_v, CH, 1.0)
    _fill1d(zs_v, NPT, 0.0)
    pltpu.sync_copy(zs_v, acc.at[pl.ds(sid * NPT, NPT)])
    plsc.subcore_barrier()

    def step(s, carry):
        pltpu.sync_copy(row_hbm.at[pl.ds(base + s * CH, CH)], idx_v)
        pltpu.sync_copy(ones_v, acc.at[idx_v], add=True)
        return carry
    lax.fori_loop(0, NSTEP, step, 0)
    plsc.subcore_barrier()
    pltpu.sync_copy(acc.at[pl.ds(sid * NPT, NPT)],
                    out_hbm.at[cid, pl.ds(sid * NPT, NPT)])


def _sc_conv(ys_hbm, row_hbm, col_hbm, zeros_hbm, out_hbm,
             rowall_v, cidx_v, rows_v, sems, csems, acc):
    cid = lax.axis_index("c")
    sid = lax.axis_index("s")
    wid = sid * NC + cid
    base = wid * EPT
    pltpu.sync_copy(row_hbm.at[pl.ds(base, EPT)], rowall_v)
    pltpu.sync_copy(zeros_hbm.at[pl.ds(sid * NPT2, NPT2), :],
                    acc.at[pl.ds(sid * NPT2, NPT2), :])
    plsc.subcore_barrier()

    def step(s4, carry):
        descs = []
        cdescs = []
        for b in range(NBUFC):
            s = s4 * NBUFC + b
            cdescs.append(pltpu.async_copy(
                col_hbm.at[pl.ds(base + s * CH, CH)], cidx_v[b], csems[b]))
            descs.append(pltpu.async_copy(
                ys_hbm.at[rowall_v.at[pl.ds(s * CH, CH)]],
                rows_v[b], sems[b]))
        for b in range(NBUFC):
            cdescs[b].wait()
            descs[b].wait()
            pltpu.sync_copy(rows_v[b], acc.at[cidx_v[b]], add=True)
        return carry
    lax.fori_loop(0, NSTEP // NBUFC, step, 0)
    plsc.subcore_barrier()
    pltpu.sync_copy(acc.at[pl.ds(sid * NPT2, NPT2), :],
                    out_hbm.at[cid, pl.ds(sid * NPT2, NPT2), :])


_sc_conv = functools.partial(
    pl.kernel,
    out_type=jax.ShapeDtypeStruct((NC, NP2, HH), F32),
    mesh=_MESH,
    compiler_params=pltpu.CompilerParams(needs_layout_passes=False),
    scratch_types=[
        pltpu.VMEM((EPT,), I32),
        [pltpu.VMEM((CH,), I32) for _ in range(NBUFC)],
        [pltpu.VMEM((CH, HH), F32) for _ in range(NBUFC)],
        [pltpu.SemaphoreType.DMA for _ in range(NBUFC)],
        [pltpu.SemaphoreType.DMA for _ in range(NBUFC)],
        pltpu.VMEM_SHARED((NP2, HH), F32),
    ],
)(_sc_conv)


@functools.partial(
    pl.kernel,
    out_type=(jax.ShapeDtypeStruct((EPAD,), F32),
              jax.ShapeDtypeStruct((NC, NP), F32)),
    mesh=_MESH,
    compiler_params=pltpu.CompilerParams(needs_layout_passes=False),
    scratch_types=[
        pltpu.VMEM((NP,), F32),
        pltpu.VMEM((NP,), F32),
        pltpu.VMEM((EPT,), I32),
        pltpu.VMEM((EPT,), I32),
        pltpu.VMEM((EPT,), F32),
        pltpu.VMEM((CH,), I32),
        pltpu.VMEM((NPT,), F32),
        pltpu.VMEM_SHARED((NP,), F32),
    ],
)
def _sc_edge_att(da_hbm, db_hbm, row_hbm, col_hbm, ew_hbm, deg_hbm,
                 da_v, db_v, row_v, col_v, ew_v, idx_v, zs_v, acc):
    cid = lax.axis_index("c")
    sid = lax.axis_index("s")
    wid = sid * NC + cid
    base = wid * EPT
    _fill1d(zs_v, NPT, 0.0)
    pltpu.sync_copy(zs_v, acc.at[pl.ds(sid * NPT, NPT)])
    pltpu.sync_copy(da_hbm, da_v.at[pl.ds(0, NN)])
    pltpu.sync_copy(db_hbm, db_v.at[pl.ds(0, NN)])
    pltpu.sync_copy(row_hbm.at[pl.ds(base, EPT)], row_v)
    pltpu.sync_copy(col_hbm.at[pl.ds(base, EPT)], col_v)
    plsc.subcore_barrier()

    def step(j, carry):
        r16 = row_v[pl.ds(j * LN, LN)]
        c16 = col_v[pl.ds(j * LN, LN)]
        va = plsc.load_gather(da_v, [r16])
        vb = plsc.load_gather(db_v, [c16])
        ew = 1.0 / (1.0 + jnp.exp(-(va + vb)))
        ew_v[pl.ds(j * LN, LN)] = ew
        return carry
    lax.fori_loop(0, EPT // LN, step, 0)
    pltpu.sync_copy(ew_v, ew_hbm.at[pl.ds(base, EPT)])

    def step2(s, carry):
        pltpu.sync_copy(row_hbm.at[pl.ds(base + s * CH, CH)], idx_v)
        pltpu.sync_copy(ew_v.at[pl.ds(s * CH, CH)], acc.at[idx_v], add=True)
        return carry
    lax.fori_loop(0, NSTEP, step2, 0)
    plsc.subcore_barrier()
    pltpu.sync_copy(acc.at[pl.ds(sid * NPT, NPT)],
                    deg_hbm.at[cid, pl.ds(sid * NPT, NPT)])


def _make_sc_wconv(is_ctx):
    def body(z_hbm, ew_hbm, row_hbm, col_hbm, zeros_hbm, q_hbm,
             rowall_v, cidx_v, ewb_v, rows_v, sems, csems, esems, acc):
        cid = lax.axis_index("c")
        sid = lax.axis_index("s")
        wid = sid * NC + cid
        base = wid * EPT
        pltpu.sync_copy(row_hbm.at[pl.ds(base, EPT)], rowall_v)
        pltpu.sync_copy(zeros_hbm.at[pl.ds(sid * NPT2, NPT2), :],
                        acc.at[pl.ds(sid * NPT2, NPT2), :])
        plsc.subcore_barrier()

        def step(s4, carry):
            descs = []
            cdescs = []
            edescs = []
            for b in range(NBUF):
                s = s4 * NBUF + b
                cdescs.append(pltpu.async_copy(
                    col_hbm.at[pl.ds(base + s * CH, CH)], cidx_v[b],
                    csems[b]))
                edescs.append(pltpu.async_copy(
                    ew_hbm.at[pl.ds(base + s * CH, CH)], ewb_v[b],
                    esems[b]))
                descs.append(pltpu.async_copy(
                    z_hbm.at[rowall_v.at[pl.ds(s * CH, CH)]],
                    rows_v[b], sems[b]))
            for b in range(NBUF):
                cdescs[b].wait()
                edescs[b].wait()
                descs[b].wait()

                def scale(i16, c2):
                    w16 = ewb_v[b][pl.ds(i16 * LN, LN)]
                    if not is_ctx:
                        w16 = 1.0 - w16
                    for l in range(LN):
                        w = w16[l]
                        r = i16 * LN + l
                        for j in range(HH // LN):
                            rows_v[b][r, pl.ds(j * LN, LN)] = (
                                w * rows_v[b][r, pl.ds(j * LN, LN)])
                    return c2
                lax.fori_loop(0, CH // LN, scale, 0)
                pltpu.sync_copy(rows_v[b], acc.at[cidx_v[b]], add=True)
            return carry
        lax.fori_loop(0, NSTEP // NBUF, step, 0)
        plsc.subcore_barrier()
        pltpu.sync_copy(acc.at[pl.ds(sid * NPT2, NPT2), :],
                        q_hbm.at[cid, pl.ds(sid * NPT2, NPT2), :])

    return pl.kernel(
        body,
        out_type=jax.ShapeDtypeStruct((NC, NP2, HH), F32),
        mesh=_MESH,
        compiler_params=pltpu.CompilerParams(needs_layout_passes=False),
        scratch_types=[
            pltpu.VMEM((EPT,), I32),
            [pltpu.VMEM((CH,), I32) for _ in range(NBUF)],
            [pltpu.VMEM((CH,), F32) for _ in range(NBUF)],
            [pltpu.VMEM((CH, HH), F32) for _ in range(NBUF)],
            [pltpu.SemaphoreType.DMA for _ in range(NBUF)],
            [pltpu.SemaphoreType.DMA for _ in range(NBUF)],
            [pltpu.SemaphoreType.DMA for _ in range(NBUF)],
            pltpu.VMEM_SHARED((NP2, HH), F32),
        ],
    )


_sc_wconv_c = _make_sc_wconv(True)
_sc_wconv_o = _make_sc_wconv(False)


def _bn(x):
    mean = jnp.mean(x, axis=0, keepdims=True)
    var = jnp.mean((x - mean) ** 2, axis=0, keepdims=True)
    return (x - mean) / jnp.sqrt(var + EPS) * 1.0 + 0.0001


def _mm(a, b):
    return lax.dot_general(a, b, (((1,), (0,)), ((), ())),
                           precision=lax.Precision.HIGHEST,
                           preferred_element_type=F32)


def _tc_call(body, out_shapes):
    return pl.pallas_call(
        body,
        out_shape=out_shapes,
    )


def _tc_feat_body(x_ref, w_ref, b_ref, h_ref):
    x = _bn(x_ref[...])
    h_ref[...] = jax.nn.relu(_mm(x, w_ref[...]) + b_ref[...][None, :])


def _tc_prep0_body(h_ref, w_ref, degp_ref, ys_ref, dinv_ref, outdeg_ref):
    outdeg = degp_ref[0, :NN] + degp_ref[1, :NN]
    dinv = lax.rsqrt(outdeg + 1.0)
    dinv_ref[...] = dinv[:, None]
    outdeg_ref[...] = outdeg[:, None]
    xb = _bn(h_ref[...])
    ys_ref[...] = dinv[:, None] * _mm(xb, w_ref[...])


def _tc_prep_body(p_ref, ysp_ref, dinv_ref, bprev_ref, w_ref, ys_ref):
    dinv = dinv_ref[...]
    out = dinv * (p_ref[0, :NN, :] + p_ref[1, :NN, :] + ysp_ref[...]) \
        + bprev_ref[...][None, :]
    h = jax.nn.relu(out)
    xb = _bn(h)
    ys_ref[...] = dinv * _mm(xb, w_ref[...])


def _tc_finish_body(p_ref, ysp_ref, dinv_ref, bprev_ref, x_ref):
    out = dinv_ref[...] * (p_ref[0, :NN, :] + p_ref[1, :NN, :]
                           + ysp_ref[...]) + bprev_ref[...][None, :]
    x_ref[...] = jax.nn.relu(out)


def _tc_edge_proj_body(x_ref, wea_ref, bea_ref, da_ref, db_ref):
    x = x_ref[...]
    wea = wea_ref[...]
    wd_a = wea[:HH, 0:1] - wea[:HH, 1:2]
    wd_b = wea[HH:, 0:1] - wea[HH:, 1:2]
    bea = bea_ref[...]
    da_ref[...] = _mm(x, wd_a) + (bea[0] - bea[1])
    db_ref[...] = _mm(x, wd_b)


def _tc_node_att_body(x_ref, wna_ref, bna_ref, xc_ref, xo_ref):
    x = x_ref[...]
    nl = _mm(x, wna_ref[...]) + bna_ref[...][None, :]
    na0 = jax.nn.sigmoid(nl[:, 0:1] - nl[:, 1:2])
    xc_ref[...] = na0 * x
    xo_ref[...] = (1.0 - na0) * x


def _make_wprep_body(is_ctx):
    def body(x_ref, w_ref, degcp_ref, outdeg_ref, z_ref, dinv_ref):
        degc = degcp_ref[0, :NN] + degcp_ref[1, :NN] + 1.0
        if is_ctx:
            deg = degc
        else:
            deg = outdeg_ref[...][:, 0] + 2.0 - degc
        dinv = lax.rsqrt(deg)[:, None]
        dinv_ref[...] = dinv
        z_ref[...] = dinv * _mm(_bn(x_ref[...]), w_ref[...])
    return body


def _readout(h, w1, b1, w2, b2):
    h = _bn(h)
    h = jax.nn.relu(_mm(h, w1) + b1[None, :])
    h = _bn(h)
    h = _mm(h, w2) + b2[None, :]
    m = jnp.max(h, axis=-1, keepdims=True)
    lse = jnp.log(jnp.sum(jnp.exp(h - m), axis=-1, keepdims=True)) + m
    return h - lse


def _tc_pool_body(q_ref, z_ref, dinv_ref, b_ref, batch_ref, p_ref):
    xr = jax.nn.relu(
        dinv_ref[...] * (q_ref[0, :NN, :] + q_ref[1, :NN, :] + z_ref[...])
        + b_ref[...][None, :])
    gi = lax.broadcasted_iota(I32, (1, GG), 1)
    oh = (batch_ref[...] == gi).astype(F32)
    p_ref[...] = lax.dot_general(oh, xr, (((0,), (0,)), ((), ())),
                                 precision=lax.Precision.HIGHEST,
                                 preferred_element_type=F32)


def _tc_readout_body(pc_ref, po_ref,
                     w1c_ref, b1c_ref, w2c_ref, b2c_ref,
                     w1o_ref, b1o_ref, w2o_ref, b2o_ref,
                     w1co_ref, b1co_ref, w2co_ref, b2co_ref,
                     lc_ref, lo_ref, lco_ref):
    pc = pc_ref[...]
    po = po_ref[...]
    lc_ref[...] = _readout(pc, w1c_ref[...], b1c_ref[...],
                           w2c_ref[...], b2c_ref[...])
    lo_ref[...] = _readout(po, w1o_ref[...], b1o_ref[...],
                           w2o_ref[...], b2o_ref[...])
    lco_ref[...] = _readout(pc + po, w1co_ref[...], b1co_ref[...],
                            w2co_ref[...], b2co_ref[...])


def kernel(x, params, edge_index, batch):
    row = edge_index[0]
    col = edge_index[1]
    pad = EPAD - EE
    row_p = jnp.concatenate([row, jnp.zeros((pad,), I32)])
    rowd_p = jnp.concatenate([row, jnp.full((pad,), DUMMY, I32)])
    col_p = jnp.concatenate([col, jnp.full((pad,), DUMMY, I32)])
    zeros2 = jnp.zeros((NP2, HH), F32)
    sds = jax.ShapeDtypeStruct

    degp = _sc_outdeg(rowd_p)

    h = _tc_call(_tc_feat_body, sds((NN, HH), F32))(
        x, params["W_feat"], params["b_feat"])

    ys, dinv, outdeg = _tc_call(
        _tc_prep0_body,
        (sds((NN, HH), F32), sds((NN, 1), F32), sds((NN, 1), F32)))(
        h, params["W_convs"][0], degp)

    for i in range(1, 4):
        part = _sc_conv(ys, row_p, col_p, zeros2)
        if i < 3:
            ys = _tc_call(_tc_prep_body, sds((NN, HH), F32))(
                part, ys, dinv, params["b_convs"][i - 1],
                params["W_convs"][i])

    xatt = _tc_call(_tc_finish_body, sds((NN, HH), F32))(
        part, ys, dinv, params["b_convs"][2])

    da, db = _tc_call(
        _tc_edge_proj_body, (sds((NN, 1), F32), sds((NN, 1), F32)))(
        xatt, params["W_edge_att"], params["b_edge_att"])
    xc, xo = _tc_call(
        _tc_node_att_body, (sds((NN, HH), F32), sds((NN, HH), F32)))(
        xatt, params["W_node_att"], params["b_node_att"])

    ewc, degcp = _sc_edge_att(da[:, 0], db[:, 0], rowd_p, col_p)

    zc, dinvc = _tc_call(
        _make_wprep_body(True), (sds((NN, HH), F32), sds((NN, 1), F32)))(
        xc, params["W_ctx"], degcp, outdeg)
    zo, dinvo = _tc_call(
        _make_wprep_body(False), (sds((NN, HH), F32), sds((NN, 1), F32)))(
        xo, params["W_obj"], degcp, outdeg)

    qc = _sc_wconv_c(zc, ewc, row_p, col_p, zeros2)
    qo = _sc_wconv_o(zo, ewc, row_p, col_p, zeros2)

    batch2 = batch[:, None]
    pool = _tc_call(_tc_pool_body, sds((GG, HH), F32))
    pc = pool(qc, zc, dinvc, params["b_ctx"], batch2)
    po = pool(qo, zo, dinvo, params["b_obj"], batch2)

    lc, lo, lco = _tc_call(
        _tc_readout_body,
        (sds((GG, CC), F32), sds((GG, CC), F32), sds((GG, CC), F32)))(
        pc, po,
        params["W_fc1_c"], params["b_fc1_c"],
        params["W_fc2_c"], params["b_fc2_c"],
        params["W_fc1_o"], params["b_fc1_o"],
        params["W_fc2_o"], params["b_fc2_o"],
        params["W_fc1_co"], params["b_fc1_co"],
        params["W_fc2_co"], params["b_fc2_co"])
    return lc, lo, lco


# speedup vs baseline: 1.1315x; 1.1092x over previous
import functools
import jax
import jax.numpy as jnp
from jax import lax
from jax.experimental import pallas as pl
from jax.experimental.pallas import tpu as pltpu
from jax.experimental.pallas import tpu_sc as plsc

NN = 10000
EE = 320000
HH = 128
CC = 10
GG = 128
EPS = 1e-5

NC, NS, LN = 2, 16, 16
NW = NC * NS
NP = 10240
NPT = NP // NS
EPT = 10240
EPAD = EPT * NW
CH = 128
NSTEP = EPT // CH
NBUF = 2
NBUFC = 2
NP2 = 10112
NPT2 = NP2 // NS
EPTA = 14336
EPTB = 6144
NSTEPA = EPTA // CH
NSTEPB = EPTB // CH
DUMMY = NN + 8

_MESH = plsc.VectorSubcoreMesh(core_axis_name="c", subcore_axis_name="s")

F32 = jnp.float32
I32 = jnp.int32


def _fill1d(ref, n, value):
    def body(j, carry):
        ref[pl.ds(j * LN, LN)] = jnp.full((LN,), value, F32)
        return carry
    lax.fori_loop(0, n // LN, body, 0)


@functools.partial(
    pl.kernel,
    out_type=jax.ShapeDtypeStruct((NC, NP), F32),
    mesh=_MESH,
    compiler_params=pltpu.CompilerParams(needs_layout_passes=False),
    scratch_types=[
        pltpu.VMEM((CH,), I32),
        pltpu.VMEM((CH,), F32),
        pltpu.VMEM((NPT,), F32),
        pltpu.VMEM_SHARED((NP,), F32),
    ],
)
def _sc_outdeg(row_hbm, out_hbm, idx_v, ones_v, zs_v, acc):
    cid = lax.axis_index("c")
    sid = lax.axis_index("s")
    wid = sid * NC + cid
    base = wid * EPT
    _fill1d(ones_v, CH, 1.0)
    _fill1d(zs_v, NPT, 0.0)
    pltpu.sync_copy(zs_v, acc.at[pl.ds(sid * NPT, NPT)])
    plsc.subcore_barrier()

    def step(s, carry):
        pltpu.sync_copy(row_hbm.at[pl.ds(base + s * CH, CH)], idx_v)
        pltpu.sync_copy(ones_v, acc.at[idx_v], add=True)
        return carry
    lax.fori_loop(0, NSTEP, step, 0)
    plsc.subcore_barrier()
    pltpu.sync_copy(acc.at[pl.ds(sid * NPT, NPT)],
                    out_hbm.at[cid, pl.ds(sid * NPT, NPT)])


def _sc_conv(ys_hbm, row_hbm, col_hbm, zeros_hbm, out_hbm,
             rowall_v, cidx_v, rows_v, sems, csems, acc):
    cid = lax.axis_index("c")
    sid = lax.axis_index("s")
    base = jnp.where(cid == 0, sid * EPTA, NS * EPTA + sid * EPTB)
    nstep = jnp.where(cid == 0, NSTEPA, NSTEPB)

    @pl.when(cid == 0)
    def _():
        pltpu.sync_copy(row_hbm.at[pl.ds(sid * EPTA, EPTA)],
                        rowall_v.at[pl.ds(0, EPTA)])

    @pl.when(cid == 1)
    def _():
        pltpu.sync_copy(row_hbm.at[pl.ds(NS * EPTA + sid * EPTB, EPTB)],
                        rowall_v.at[pl.ds(0, EPTB)])
    pltpu.sync_copy(zeros_hbm.at[pl.ds(sid * NPT2, NPT2), :],
                    acc.at[pl.ds(sid * NPT2, NPT2), :])
    plsc.subcore_barrier()

    def step(s4, carry):
        descs = []
        cdescs = []
        for b in range(NBUFC):
            s = s4 * NBUFC + b
            cdescs.append(pltpu.async_copy(
                col_hbm.at[pl.ds(base + s * CH, CH)], cidx_v[b], csems[b]))
            descs.append(pltpu.async_copy(
                ys_hbm.at[rowall_v.at[pl.ds(s * CH, CH)]],
                rows_v[b], sems[b]))
        for b in range(NBUFC):
            cdescs[b].wait()
            descs[b].wait()
            pltpu.sync_copy(rows_v[b], acc.at[cidx_v[b]], add=True)
        return carry
    lax.fori_loop(0, nstep // NBUFC, step, 0)
    plsc.subcore_barrier()
    pltpu.sync_copy(acc.at[pl.ds(sid * NPT2, NPT2), :],
                    out_hbm.at[cid, pl.ds(sid * NPT2, NPT2), :])


_sc_conv = functools.partial(
    pl.kernel,
    out_type=jax.ShapeDtypeStruct((NC, NP2, HH), F32),
    mesh=_MESH,
    compiler_params=pltpu.CompilerParams(needs_layout_passes=False),
    scratch_types=[
        pltpu.VMEM((EPTA,), I32),
        [pltpu.VMEM((CH,), I32) for _ in range(NBUFC)],
        [pltpu.VMEM((CH, HH), F32) for _ in range(NBUFC)],
        [pltpu.SemaphoreType.DMA for _ in range(NBUFC)],
        [pltpu.SemaphoreType.DMA for _ in range(NBUFC)],
        pltpu.VMEM_SHARED((NP2, HH), F32),
    ],
)(_sc_conv)


@functools.partial(
    pl.kernel,
    out_type=(jax.ShapeDtypeStruct((EPAD,), F32),
              jax.ShapeDtypeStruct((NC, NP), F32)),
    mesh=_MESH,
    compiler_params=pltpu.CompilerParams(needs_layout_passes=False),
    scratch_types=[
        pltpu.VMEM((NP,), F32),
        pltpu.VMEM((NP,), F32),
        pltpu.VMEM((EPT,), I32),
        pltpu.VMEM((EPT,), I32),
        pltpu.VMEM((EPT,), F32),
        pltpu.VMEM((CH,), I32),
        pltpu.VMEM((NPT,), F32),
        pltpu.VMEM_SHARED((NP,), F32),
    ],
)
def _sc_edge_att(da_hbm, db_hbm, row_hbm, col_hbm, ew_hbm, deg_hbm,
                 da_v, db_v, row_v, col_v, ew_v, idx_v, zs_v, acc):
    cid = lax.axis_index("c")
    sid = lax.axis_index("s")
    wid = sid * NC + cid
    base = wid * EPT
    _fill1d(zs_v, NPT, 0.0)
    pltpu.sync_copy(zs_v, acc.at[pl.ds(sid * NPT, NPT)])
    pltpu.sync_copy(da_hbm, da_v.at[pl.ds(0, NN)])
    pltpu.sync_copy(db_hbm, db_v.at[pl.ds(0, NN)])
    pltpu.sync_copy(row_hbm.at[pl.ds(base, EPT)], row_v)
    pltpu.sync_copy(col_hbm.at[pl.ds(base, EPT)], col_v)
    plsc.subcore_barrier()

    def step(j, carry):
        r16 = row_v[pl.ds(j * LN, LN)]
        c16 = col_v[pl.ds(j * LN, LN)]
        va = plsc.load_gather(da_v, [r16])
        vb = plsc.load_gather(db_v, [c16])
        ew = 1.0 / (1.0 + jnp.exp(-(va + vb)))
        ew_v[pl.ds(j * LN, LN)] = ew
        return carry
    lax.fori_loop(0, EPT // LN, step, 0)
    pltpu.sync_copy(ew_v, ew_hbm.at[pl.ds(base, EPT)])

    def step2(s, carry):
        pltpu.sync_copy(row_hbm.at[pl.ds(base + s * CH, CH)], idx_v)
        pltpu.sync_copy(ew_v.at[pl.ds(s * CH, CH)], acc.at[idx_v], add=True)
        return carry
    lax.fori_loop(0, NSTEP, step2, 0)
    plsc.subcore_barrier()
    pltpu.sync_copy(acc.at[pl.ds(sid * NPT, NPT)],
                    deg_hbm.at[cid, pl.ds(sid * NPT, NPT)])


def _make_sc_wconv(is_ctx):
    def body(z_hbm, ew_hbm, row_hbm, col_hbm, zeros_hbm, q_hbm,
             rowall_v, cidx_v, ewb_v, rows_v, sems, csems, esems, acc):
        cid = lax.axis_index("c")
        sid = lax.axis_index("s")
        base = jnp.where(cid == 0, sid * EPTA, NS * EPTA + sid * EPTB)
        nstep = jnp.where(cid == 0, NSTEPA, NSTEPB)

        @pl.when(cid == 0)
        def _():
            pltpu.sync_copy(row_hbm.at[pl.ds(sid * EPTA, EPTA)],
                            rowall_v.at[pl.ds(0, EPTA)])

        @pl.when(cid == 1)
        def _():
            pltpu.sync_copy(row_hbm.at[pl.ds(NS * EPTA + sid * EPTB, EPTB)],
                            rowall_v.at[pl.ds(0, EPTB)])
        pltpu.sync_copy(zeros_hbm.at[pl.ds(sid * NPT2, NPT2), :],
                        acc.at[pl.ds(sid * NPT2, NPT2), :])
        plsc.subcore_barrier()

        def step(s4, carry):
            descs = []
            cdescs = []
            edescs = []
            for b in range(NBUF):
                s = s4 * NBUF + b
                cdescs.append(pltpu.async_copy(
                    col_hbm.at[pl.ds(base + s * CH, CH)], cidx_v[b],
                    csems[b]))
                edescs.append(pltpu.async_copy(
                    ew_hbm.at[pl.ds(base + s * CH, CH)], ewb_v[b],
                    esems[b]))
                descs.append(pltpu.async_copy(
                    z_hbm.at[rowall_v.at[pl.ds(s * CH, CH)]],
                    rows_v[b], sems[b]))
            for b in range(NBUF):
                cdescs[b].wait()
                edescs[b].wait()
                descs[b].wait()

                def scale(i16, c2):
                    w16 = ewb_v[b][pl.ds(i16 * LN, LN)]
                    if not is_ctx:
                        w16 = 1.0 - w16
                    for l in range(LN):
                        w = w16[l]
                        r = i16 * LN + l
                        for j in range(HH // LN):
                            rows_v[b][r, pl.ds(j * LN, LN)] = (
                                w * rows_v[b][r, pl.ds(j * LN, LN)])
                    return c2
                lax.fori_loop(0, CH // LN, scale, 0)
                pltpu.sync_copy(rows_v[b], acc.at[cidx_v[b]], add=True)
            return carry
        lax.fori_loop(0, nstep // NBUF, step, 0)
        plsc.subcore_barrier()
        pltpu.sync_copy(acc.at[pl.ds(sid * NPT2, NPT2), :],
                        q_hbm.at[cid, pl.ds(sid * NPT2, NPT2), :])

    return pl.kernel(
        body,
        out_type=jax.ShapeDtypeStruct((NC, NP2, HH), F32),
        mesh=_MESH,
        compiler_params=pltpu.CompilerParams(needs_layout_passes=False),
        scratch_types=[
            pltpu.VMEM((EPTA,), I32),
            [pltpu.VMEM((CH,), I32) for _ in range(NBUF)],
            [pltpu.VMEM((CH,), F32) for _ in range(NBUF)],
            [pltpu.VMEM((CH, HH), F32) for _ in range(NBUF)],
            [pltpu.SemaphoreType.DMA for _ in range(NBUF)],
            [pltpu.SemaphoreType.DMA for _ in range(NBUF)],
            [pltpu.SemaphoreType.DMA for _ in range(NBUF)],
            pltpu.VMEM_SHARED((NP2, HH), F32),
        ],
    )


_sc_wconv_c = _make_sc_wconv(True)
_sc_wconv_o = _make_sc_wconv(False)


def _bn(x):
    mean = jnp.mean(x, axis=0, keepdims=True)
    var = jnp.mean((x - mean) ** 2, axis=0, keepdims=True)
    return (x - mean) / jnp.sqrt(var + EPS) * 1.0 + 0.0001


def _mm(a, b):
    return lax.dot_general(a, b, (((1,), (0,)), ((), ())),
                           precision=lax.Precision.HIGHEST,
                           preferred_element_type=F32)


def _tc_call(body, out_shapes):
    return pl.pallas_call(
        body,
        out_shape=out_shapes,
    )


def _tc_feat_body(x_ref, w_ref, b_ref, h_ref):
    x = _bn(x_ref[...])
    h_ref[...] = jax.nn.relu(_mm(x, w_ref[...]) + b_ref[...][None, :])


def _tc_prep0_body(h_ref, w_ref, degp_ref, ys_ref, dinv_ref, outdeg_ref):
    outdeg = degp_ref[0, :NN] + degp_ref[1, :NN]
    dinv = lax.rsqrt(outdeg + 1.0)
    dinv_ref[...] = dinv[:, None]
    outdeg_ref[...] = outdeg[:, None]
    xb = _bn(h_ref[...])
    ys_ref[...] = dinv[:, None] * _mm(xb, w_ref[...])


def _tc_prep_body(p_ref, ysp_ref, dinv_ref, bprev_ref, w_ref, ys_ref):
    dinv = dinv_ref[...]
    out = dinv * (p_ref[0, :NN, :] + p_ref[1, :NN, :] + ysp_ref[...]) \
        + bprev_ref[...][None, :]
    h = jax.nn.relu(out)
    xb = _bn(h)
    ys_ref[...] = dinv * _mm(xb, w_ref[...])


def _tc_finish_body(p_ref, ysp_ref, dinv_ref, bprev_ref, x_ref):
    out = dinv_ref[...] * (p_ref[0, :NN, :] + p_ref[1, :NN, :]
                           + ysp_ref[...]) + bprev_ref[...][None, :]
    x_ref[...] = jax.nn.relu(out)


def _tc_edge_proj_body(x_ref, wea_ref, bea_ref, da_ref, db_ref):
    x = x_ref[...]
    wea = wea_ref[...]
    wd_a = wea[:HH, 0:1] - wea[:HH, 1:2]
    wd_b = wea[HH:, 0:1] - wea[HH:, 1:2]
    bea = bea_ref[...]
    da_ref[...] = _mm(x, wd_a) + (bea[0] - bea[1])
    db_ref[...] = _mm(x, wd_b)


def _tc_node_att_body(x_ref, wna_ref, bna_ref, xc_ref, xo_ref):
    x = x_ref[...]
    nl = _mm(x, wna_ref[...]) + bna_ref[...][None, :]
    na0 = jax.nn.sigmoid(nl[:, 0:1] - nl[:, 1:2])
    xc_ref[...] = na0 * x
    xo_ref[...] = (1.0 - na0) * x


def _make_wprep_body(is_ctx):
    def body(x_ref, w_ref, degcp_ref, outdeg_ref, z_ref, dinv_ref):
        degc = degcp_ref[0, :NN] + degcp_ref[1, :NN] + 1.0
        if is_ctx:
            deg = degc
        else:
            deg = outdeg_ref[...][:, 0] + 2.0 - degc
        dinv = lax.rsqrt(deg)[:, None]
        dinv_ref[...] = dinv
        z_ref[...] = dinv * _mm(_bn(x_ref[...]), w_ref[...])
    return body


def _readout(h, w1, b1, w2, b2):
    h = _bn(h)
    h = jax.nn.relu(_mm(h, w1) + b1[None, :])
    h = _bn(h)
    h = _mm(h, w2) + b2[None, :]
    m = jnp.max(h, axis=-1, keepdims=True)
    lse = jnp.log(jnp.sum(jnp.exp(h - m), axis=-1, keepdims=True)) + m
    return h - lse


def _tc_pool_body(q_ref, z_ref, dinv_ref, b_ref, batch_ref, p_ref):
    xr = jax.nn.relu(
        dinv_ref[...] * (q_ref[0, :NN, :] + q_ref[1, :NN, :] + z_ref[...])
        + b_ref[...][None, :])
    gi = lax.broadcasted_iota(I32, (1, GG), 1)
    oh = (batch_ref[...] == gi).astype(F32)
    p_ref[...] = lax.dot_general(oh, xr, (((0,), (0,)), ((), ())),
                                 precision=lax.Precision.HIGHEST,
                                 preferred_element_type=F32)


def _tc_readout_body(pc_ref, po_ref,
                     w1c_ref, b1c_ref, w2c_ref, b2c_ref,
                     w1o_ref, b1o_ref, w2o_ref, b2o_ref,
                     w1co_ref, b1co_ref, w2co_ref, b2co_ref,
                     lc_ref, lo_ref, lco_ref):
    pc = pc_ref[...]
    po = po_ref[...]
    lc_ref[...] = _readout(pc, w1c_ref[...], b1c_ref[...],
                           w2c_ref[...], b2c_ref[...])
    lo_ref[...] = _readout(po, w1o_ref[...], b1o_ref[...],
                           w2o_ref[...], b2o_ref[...])
    lco_ref[...] = _readout(pc + po, w1co_ref[...], b1co_ref[...],
                            w2co_ref[...], b2co_ref[...])


def kernel(x, params, edge_index, batch):
    row = edge_index[0]
    col = edge_index[1]
    pad = EPAD - EE
    row_p = jnp.concatenate([row, jnp.zeros((pad,), I32)])
    rowd_p = jnp.concatenate([row, jnp.full((pad,), DUMMY, I32)])
    col_p = jnp.concatenate([col, jnp.full((pad,), DUMMY, I32)])
    zeros2 = jnp.zeros((NP2, HH), F32)
    sds = jax.ShapeDtypeStruct

    degp = _sc_outdeg(rowd_p)

    h = _tc_call(_tc_feat_body, sds((NN, HH), F32))(
        x, params["W_feat"], params["b_feat"])

    ys, dinv, outdeg = _tc_call(
        _tc_prep0_body,
        (sds((NN, HH), F32), sds((NN, 1), F32), sds((NN, 1), F32)))(
        h, params["W_convs"][0], degp)

    for i in range(1, 4):
        part = _sc_conv(ys, row_p, col_p, zeros2)
        if i < 3:
            ys = _tc_call(_tc_prep_body, sds((NN, HH), F32))(
                part, ys, dinv, params["b_convs"][i - 1],
                params["W_convs"][i])

    xatt = _tc_call(_tc_finish_body, sds((NN, HH), F32))(
        part, ys, dinv, params["b_convs"][2])

    da, db = _tc_call(
        _tc_edge_proj_body, (sds((NN, 1), F32), sds((NN, 1), F32)))(
        xatt, params["W_edge_att"], params["b_edge_att"])
    xc, xo = _tc_call(
        _tc_node_att_body, (sds((NN, HH), F32), sds((NN, HH), F32)))(
        xatt, params["W_node_att"], params["b_node_att"])

    ewc, degcp = _sc_edge_att(da[:, 0], db[:, 0], rowd_p, col_p)

    zc, dinvc = _tc_call(
        _make_wprep_body(True), (sds((NN, HH), F32), sds((NN, 1), F32)))(
        xc, params["W_ctx"], degcp, outdeg)
    zo, dinvo = _tc_call(
        _make_wprep_body(False), (sds((NN, HH), F32), sds((NN, 1), F32)))(
        xo, params["W_obj"], degcp, outdeg)

    qc = _sc_wconv_c(zc, ewc, row_p, col_p, zeros2)
    qo = _sc_wconv_o(zo, ewc, row_p, col_p, zeros2)

    batch2 = batch[:, None]
    pool = _tc_call(_tc_pool_body, sds((GG, HH), F32))
    pc = pool(qc, zc, dinvc, params["b_ctx"], batch2)
    po = pool(qo, zo, dinvo, params["b_obj"], batch2)

    lc, lo, lco = _tc_call(
        _tc_readout_body,
        (sds((GG, CC), F32), sds((GG, CC), F32), sds((GG, CC), F32)))(
        pc, po,
        params["W_fc1_c"], params["b_fc1_c"],
        params["W_fc2_c"], params["b_fc2_c"],
        params["W_fc1_o"], params["b_fc1_o"],
        params["W_fc2_o"], params["b_fc2_o"],
        params["W_fc1_co"], params["b_fc1_co"],
        params["W_fc2_co"], params["b_fc2_co"])
    return lc, lo, lco
